# baseline (speedup 1.0000x reference)
import functools

import jax
import jax.numpy as jnp
from jax.experimental import pallas as pl
from jax.experimental.pallas import tpu as pltpu

_G = 128


def _gru_kernel(tok_ref, emb_ref, wit_f_ref, wht_f_ref, tb_f_ref, nb_f_ref,
                wit_b_ref, wht_b_ref, tb_b_ref, nb_b_ref, out_ref, *, T, H, V):
    blk = tok_ref.shape[0]
    tok = tok_ref[...]
    lengths = jnp.sum((tok != 0).astype(jnp.int32), axis=1, keepdims=True)
    emb = emb_ref[...]
    iota = jax.lax.broadcasted_iota(jnp.int32, (blk, V), 1)

    def make_tab(wit_ref, tb_ref):
        return (jnp.dot(emb, wit_ref[...],
                        preferred_element_type=jnp.float32)
                + tb_ref[...]).astype(jnp.bfloat16)

    tab_f = make_tab(wit_f_ref, tb_f_ref)
    tab_b = make_tab(wit_b_ref, tb_b_ref)
    wht_f = wht_f_ref[...].astype(jnp.bfloat16)
    wht_b = wht_b_ref[...].astype(jnp.bfloat16)
    nb_f = nb_f_ref[...].astype(jnp.bfloat16)
    nb_b = nb_b_ref[...].astype(jnp.bfloat16)

    def step(h, k, tab, wht, nb, mask):
        gx = jnp.dot(ohs[k], tab,
                     preferred_element_type=jnp.float32).astype(jnp.bfloat16)
        gh = jnp.dot(h, wht,
                     preferred_element_type=jnp.float32).astype(jnp.bfloat16)
        ta = jnp.tanh(gx[:, :_G] + gh[:, :_G])
        tb = jnp.tanh(gx[:, _G:2 * _G] + gh[:, _G:2 * _G])
        ghn = gh[:, 2 * _G:] + nb
        n = jnp.tanh(gx[:, 2 * _G:] + ghn + ta * ghn)
        h_new = jnp.bfloat16(0.5) * ((n + h) + tb * (h - n))
        return jnp.where(mask, h_new, h)

    masks = [jnp.broadcast_to(k < lengths, (blk, _G)) for k in range(T)]
    ohs = [(tok[:, k:k + 1] == iota).astype(jnp.bfloat16) for k in range(T)]
    hf = jnp.zeros((blk, _G), jnp.bfloat16)
    hb = jnp.zeros((blk, _G), jnp.bfloat16)
    for k in range(T):
        hf = step(hf, k, tab_f, wht_f, nb_f, masks[k])
        hb = step(hb, T - 1 - k, tab_b, wht_b, nb_b, masks[T - 1 - k])
    out_ref[...] = jnp.concatenate([hf[:, :H], hb[:, :H]],
                                   axis=1).astype(jnp.float32)


def _pack_w(W, H, scales):
    K = W.shape[1]
    s = jnp.asarray(scales, W.dtype).reshape(3, 1, 1)
    W3 = jnp.pad(W.reshape(3, H, K) * s, ((0, 0), (0, _G - H), (0, 0)))
    return W3.reshape(3 * _G, K).T


def _pack_b(b, H, scales):
    s = jnp.asarray(scales, b.dtype).reshape(3, 1)
    return jnp.pad(b.reshape(3, H) * s,
                   ((0, 0), (0, _G - H))).reshape(1, 3 * _G)


def kernel(c, q, ch, qh, char_emb, Wi_f, Wh_f, bi_f, bh_f,
           Wi_b, Wh_b, bi_b, bh_b):
    T = ch.shape[2]
    N1 = ch.shape[0] * ch.shape[1]
    N2 = qh.shape[0] * qh.shape[1]
    H = Wh_f.shape[1]
    V = char_emb.shape[0]
    tokens = jnp.concatenate(
        [ch.reshape(N1, T), qh.reshape(N2, T)], axis=0).astype(jnp.int32)
    N = N1 + N2

    blk = 800
    npad = (-N) % blk
    if npad:
        tokens = jnp.pad(tokens, ((0, npad), (0, 0)))
    ntot = N + npad

    si = (0.5, 0.5, 1.0)
    sh = (0.5, 0.5, 0.5)
    wit_f = _pack_w(Wi_f, H, si)
    wit_b = _pack_w(Wi_b, H, si)
    wht_f = jnp.pad(_pack_w(Wh_f, H, sh), ((0, _G - H), (0, 0)))
    wht_b = jnp.pad(_pack_w(Wh_b, H, sh), ((0, _G - H), (0, 0)))
    tbias_f = _pack_b(bi_f, H, si) + _pack_b(bh_f, H, (0.5, 0.5, 0.0))
    tbias_b = _pack_b(bi_b, H, si) + _pack_b(bh_b, H, (0.5, 0.5, 0.0))
    nb_f = _pack_b(bh_f, H, sh)[:, 2 * _G:]
    nb_b = _pack_b(bh_b, H, sh)[:, 2 * _G:]

    full = lambda a: pl.BlockSpec(a.shape, lambda i: (0,) * a.ndim)
    out = pl.pallas_call(
        functools.partial(_gru_kernel, T=T, H=H, V=V),
        grid=(ntot // blk,),
        in_specs=[
            pl.BlockSpec((blk, T), lambda i: (i, 0)),
            full(char_emb), full(wit_f), full(wht_f), full(tbias_f),
            full(nb_f), full(wit_b), full(wht_b), full(tbias_b), full(nb_b),
        ],
        out_specs=pl.BlockSpec((blk, 2 * H), lambda i: (i, 0)),
        out_shape=jax.ShapeDtypeStruct((ntot, 2 * H), jnp.float32),
        compiler_params=pltpu.CompilerParams(
            dimension_semantics=("parallel",)),
    )(tokens, char_emb, wit_f, wht_f, tbias_f, nb_f,
      wit_b, wht_b, tbias_b, nb_b)
    return out[:N1], out[N1:N]


# speedup vs baseline: 1.0116x; 1.0116x over previous
import functools

import jax
import jax.numpy as jnp
from jax.experimental import pallas as pl
from jax.experimental.pallas import tpu as pltpu

_G = 128


def _gru_kernel(tok_ref, emb_ref, wit_f_ref, wht_f_ref, tb_f_ref, nb_f_ref,
                wit_b_ref, wht_b_ref, tb_b_ref, nb_b_ref, out_ref, *, T, H, V):
    blk = tok_ref.shape[0]
    tok = tok_ref[...]
    lengths = jnp.sum((tok != 0).astype(jnp.int32), axis=1, keepdims=True)
    emb = emb_ref[...]
    iota = jax.lax.broadcasted_iota(jnp.int32, (blk, V), 1)

    def make_tab(wit_ref, tb_ref):
        return (jnp.dot(emb, wit_ref[...],
                        preferred_element_type=jnp.float32)
                + tb_ref[...]).astype(jnp.bfloat16)

    tab_f = make_tab(wit_f_ref, tb_f_ref)
    tab_b = make_tab(wit_b_ref, tb_b_ref)
    wht_f = wht_f_ref[...].astype(jnp.bfloat16)
    wht_b = wht_b_ref[...].astype(jnp.bfloat16)
    nb_f = nb_f_ref[...]
    nb_b = nb_b_ref[...]

    def step(h, k, tab, wht, nb, mask):
        gx = jnp.dot(ohs[k], tab, preferred_element_type=jnp.float32)
        gh = jnp.dot(h.astype(jnp.bfloat16), wht,
                     preferred_element_type=jnp.float32)
        ta = jnp.tanh(gx[:, :_G] + gh[:, :_G])
        tb = jnp.tanh(gx[:, _G:2 * _G] + gh[:, _G:2 * _G])
        ghn = gh[:, 2 * _G:] + nb
        n = jnp.tanh(gx[:, 2 * _G:] + ghn + ta * ghn)
        h_new = 0.5 * ((n + h) + tb * (h - n))
        return jnp.where(mask, h_new, h)

    masks = [jnp.broadcast_to(k < lengths, (blk, _G)) for k in range(T)]
    ohs = [(tok[:, k:k + 1] == iota).astype(jnp.bfloat16) for k in range(T)]
    hf = jnp.zeros((blk, _G), jnp.float32)
    hb = jnp.zeros((blk, _G), jnp.float32)
    for k in range(T):
        hf = step(hf, k, tab_f, wht_f, nb_f, masks[k])
        hb = step(hb, T - 1 - k, tab_b, wht_b, nb_b, masks[T - 1 - k])
    out_ref[...] = jnp.concatenate([hf[:, :H], hb[:, :H]], axis=1)


def _pack_w(W, H, scales):
    K = W.shape[1]
    s = jnp.asarray(scales, W.dtype).reshape(3, 1, 1)
    W3 = jnp.pad(W.reshape(3, H, K) * s, ((0, 0), (0, _G - H), (0, 0)))
    return W3.reshape(3 * _G, K).T


def _pack_b(b, H, scales):
    s = jnp.asarray(scales, b.dtype).reshape(3, 1)
    return jnp.pad(b.reshape(3, H) * s,
                   ((0, 0), (0, _G - H))).reshape(1, 3 * _G)


def kernel(c, q, ch, qh, char_emb, Wi_f, Wh_f, bi_f, bh_f,
           Wi_b, Wh_b, bi_b, bh_b):
    T = ch.shape[2]
    N1 = ch.shape[0] * ch.shape[1]
    N2 = qh.shape[0] * qh.shape[1]
    H = Wh_f.shape[1]
    V = char_emb.shape[0]
    tokens = jnp.concatenate(
        [ch.reshape(N1, T), qh.reshape(N2, T)], axis=0).astype(jnp.int32)
    N = N1 + N2

    blk = 800
    npad = (-N) % blk
    if npad:
        tokens = jnp.pad(tokens, ((0, npad), (0, 0)))
    ntot = N + npad

    si = (0.5, 0.5, 1.0)
    sh = (0.5, 0.5, 0.5)
    wit_f = _pack_w(Wi_f, H, si)
    wit_b = _pack_w(Wi_b, H, si)
    wht_f = jnp.pad(_pack_w(Wh_f, H, sh), ((0, _G - H), (0, 0)))
    wht_b = jnp.pad(_pack_w(Wh_b, H, sh), ((0, _G - H), (0, 0)))
    tbias_f = _pack_b(bi_f, H, si) + _pack_b(bh_f, H, (0.5, 0.5, 0.0))
    tbias_b = _pack_b(bi_b, H, si) + _pack_b(bh_b, H, (0.5, 0.5, 0.0))
    nb_f = _pack_b(bh_f, H, sh)[:, 2 * _G:]
    nb_b = _pack_b(bh_b, H, sh)[:, 2 * _G:]

    full = lambda a: pl.BlockSpec(a.shape, lambda i: (0,) * a.ndim)
    out = pl.pallas_call(
        functools.partial(_gru_kernel, T=T, H=H, V=V),
        grid=(ntot // blk,),
        in_specs=[
            pl.BlockSpec((blk, T), lambda i: (i, 0)),
            full(char_emb), full(wit_f), full(wht_f), full(tbias_f),
            full(nb_f), full(wit_b), full(wht_b), full(tbias_b), full(nb_b),
        ],
        out_specs=pl.BlockSpec((blk, 2 * H), lambda i: (i, 0)),
        out_shape=jax.ShapeDtypeStruct((ntot, 2 * H), jnp.float32),
        compiler_params=pltpu.CompilerParams(
            dimension_semantics=("parallel",)),
    )(tokens, char_emb, wit_f, wht_f, tbias_f, nb_f,
      wit_b, wht_b, tbias_b, nb_b)
    return out[:N1], out[N1:N]


# speedup vs baseline: 1.0473x; 1.0353x over previous
import functools

import jax
import jax.numpy as jnp
from jax.experimental import pallas as pl
from jax.experimental.pallas import tpu as pltpu

_G = 128


def _gru_kernel(tok_ref, emb_ref, wit_f_ref, wht_f_ref, tb_f_ref, nb_f_ref,
                wit_b_ref, wht_b_ref, tb_b_ref, nb_b_ref, out_ref, *, T, H, V):
    blk = tok_ref.shape[0]
    tok = tok_ref[...]
    lengths = jnp.sum((tok != 0).astype(jnp.int32), axis=1, keepdims=True)
    emb = emb_ref[...]
    iota = jax.lax.broadcasted_iota(jnp.int32, (blk, V), 1)

    def make_tab(wit_ref, tb_ref):
        return (jnp.dot(emb, wit_ref[...],
                        preferred_element_type=jnp.float32)
                + tb_ref[...]).astype(jnp.bfloat16)

    tab_f = make_tab(wit_f_ref, tb_f_ref)
    tab_b = make_tab(wit_b_ref, tb_b_ref)
    wht_f = wht_f_ref[...].astype(jnp.bfloat16)
    wht_b = wht_b_ref[...].astype(jnp.bfloat16)
    nb_f = nb_f_ref[...]
    nb_b = nb_b_ref[...]

    def step(h, k, tab, wht, nb, mask):
        gx = jnp.dot(ohs[k], tab, preferred_element_type=jnp.float32)
        gh = jnp.dot(h.astype(jnp.bfloat16), wht,
                     preferred_element_type=jnp.float32)
        ta = jnp.tanh(gx[:, :_G] + gh[:, :_G])
        tb = jnp.tanh(gx[:, _G:2 * _G] + gh[:, _G:2 * _G])
        ghn = gh[:, 2 * _G:] + nb
        n = jnp.tanh(gx[:, 2 * _G:] + ghn + ta * ghn)
        h_new = 0.5 * ((n + h) + tb * (h - n))
        return jnp.where(mask, h_new, h)

    masks = [jnp.broadcast_to(k < lengths, (blk, _G)) for k in range(T)]
    ohs = [(tok[:, k:k + 1] == iota).astype(jnp.bfloat16) for k in range(T)]
    hf = jnp.zeros((blk, _G), jnp.float32)
    hb = jnp.zeros((blk, _G), jnp.float32)
    for k in range(T):
        hf = step(hf, k, tab_f, wht_f, nb_f, masks[k])
        hb = step(hb, T - 1 - k, tab_b, wht_b, nb_b, masks[T - 1 - k])
    out_ref[...] = jnp.concatenate([hf[:, :H], hb[:, :H]], axis=1)


def _pack_w(W, H, scales):
    K = W.shape[1]
    s = jnp.asarray(scales, W.dtype).reshape(3, 1, 1)
    W3 = jnp.pad(W.reshape(3, H, K) * s, ((0, 0), (0, _G - H), (0, 0)))
    return W3.reshape(3 * _G, K).T


def _pack_b(b, H, scales):
    s = jnp.asarray(scales, b.dtype).reshape(3, 1)
    return jnp.pad(b.reshape(3, H) * s,
                   ((0, 0), (0, _G - H))).reshape(1, 3 * _G)


def kernel(c, q, ch, qh, char_emb, Wi_f, Wh_f, bi_f, bh_f,
           Wi_b, Wh_b, bi_b, bh_b):
    T = ch.shape[2]
    N1 = ch.shape[0] * ch.shape[1]
    N2 = qh.shape[0] * qh.shape[1]
    H = Wh_f.shape[1]
    V = char_emb.shape[0]

    si = (0.5, 0.5, 1.0)
    sh = (0.5, 0.5, 0.5)
    wit_f = _pack_w(Wi_f, H, si)
    wit_b = _pack_w(Wi_b, H, si)
    wht_f = jnp.pad(_pack_w(Wh_f, H, sh), ((0, _G - H), (0, 0)))
    wht_b = jnp.pad(_pack_w(Wh_b, H, sh), ((0, _G - H), (0, 0)))
    tbias_f = _pack_b(bi_f, H, si) + _pack_b(bh_f, H, (0.5, 0.5, 0.0))
    tbias_b = _pack_b(bi_b, H, si) + _pack_b(bh_b, H, (0.5, 0.5, 0.0))
    nb_f = _pack_b(bh_f, H, sh)[:, 2 * _G:]
    nb_b = _pack_b(bh_b, H, sh)[:, 2 * _G:]

    full = lambda a: pl.BlockSpec(a.shape, lambda i: (0,) * a.ndim)

    def run(tokens, blk):
        n = tokens.shape[0]
        return pl.pallas_call(
            functools.partial(_gru_kernel, T=T, H=H, V=V),
            grid=(n // blk,),
            in_specs=[
                pl.BlockSpec((blk, T), lambda i: (i, 0)),
                full(char_emb), full(wit_f), full(wht_f), full(tbias_f),
                full(nb_f), full(wit_b), full(wht_b), full(tbias_b),
                full(nb_b),
            ],
            out_specs=pl.BlockSpec((blk, 2 * H), lambda i: (i, 0)),
            out_shape=jax.ShapeDtypeStruct((n, 2 * H), jnp.float32),
            compiler_params=pltpu.CompilerParams(
                dimension_semantics=("parallel",)),
        )(tokens, char_emb, wit_f, wht_f, tbias_f, nb_f,
          wit_b, wht_b, tbias_b, nb_b)

    out1 = run(ch.reshape(N1, T).astype(jnp.int32), 800)
    out2 = run(qh.reshape(N2, T).astype(jnp.int32), 800)
    return out1, out2
